# Initial kernel scaffold; baseline (speedup 1.0000x reference)
import jax
import jax.numpy as jnp
from jax.experimental import pallas as pl


def kernel(value, depth, position, src_table, depth_table, spatial_tables):
    raise NotImplementedError("write your pallas kernel here")


# speedup vs baseline: 1.5289x; 1.5289x over previous
import functools

import jax
import jax.numpy as jnp
from jax import lax
from jax.experimental import pallas as pl
from jax.experimental.pallas import tpu as pltpu
from jax.experimental.pallas import tpu_sc as plsc

NC = 2
NS = 16
NW = NC * NS
D = 64
CHUNK = 128


def _body(value, depth, p0, p1, p2, src_tbl, dep_tbl, sp0, sp1, sp2, out,
          iv, idp, i0, i1, i2, b0, b1, b2, b3, b4, sem):
    wid = lax.axis_index("s") * NC + lax.axis_index("c")
    n_chunks = iv.shape[0]

    pltpu.sync_copy(value.at[wid], iv)
    pltpu.sync_copy(depth.at[wid], idp)
    pltpu.sync_copy(p0.at[wid], i0)
    pltpu.sync_copy(p1.at[wid], i1)
    pltpu.sync_copy(p2.at[wid], i2)

    @pl.loop(0, n_chunks)
    def _chunk(k):
        d0 = pltpu.async_copy(src_tbl.at[iv.at[k]], b0, sem)
        d1 = pltpu.async_copy(dep_tbl.at[idp.at[k]], b1, sem)
        d2 = pltpu.async_copy(sp0.at[i0.at[k]], b2, sem)
        d3 = pltpu.async_copy(sp1.at[i1.at[k]], b3, sem)
        d4 = pltpu.async_copy(sp2.at[i2.at[k]], b4, sem)
        d0.wait()
        d1.wait()
        d2.wait()
        d3.wait()
        d4.wait()

        @pl.loop(0, CHUNK)
        def _row(r):
            for c in range(D // 16):
                s = pl.ds(c * 16, 16)
                b0[r, s] = (b0[r, s] + b1[r, s]) + (b2[r, s] + b3[r, s]) \
                    + b4[r, s]

        base = (wid * n_chunks + k) * CHUNK
        pltpu.sync_copy(b0, out.at[pl.ds(base, CHUNK)])


@jax.jit
def _run(value, depth, p0, p1, p2, src_table, depth_table, sp0, sp1, sp2):
    n_tok = value.size
    n_chunks = n_tok // (NW * CHUNK)
    shape3 = (NW, n_chunks, CHUNK)
    mesh = plsc.VectorSubcoreMesh(core_axis_name="c", subcore_axis_name="s")
    f = pl.kernel(
        _body,
        out_type=jax.ShapeDtypeStruct((n_tok, D), jnp.float32),
        mesh=mesh,
        scratch_types=[
            pltpu.VMEM((n_chunks, CHUNK), jnp.int32),
            pltpu.VMEM((n_chunks, CHUNK), jnp.int32),
            pltpu.VMEM((n_chunks, CHUNK), jnp.int32),
            pltpu.VMEM((n_chunks, CHUNK), jnp.int32),
            pltpu.VMEM((n_chunks, CHUNK), jnp.int32),
            pltpu.VMEM((CHUNK, D), jnp.float32),
            pltpu.VMEM((CHUNK, D), jnp.float32),
            pltpu.VMEM((CHUNK, D), jnp.float32),
            pltpu.VMEM((CHUNK, D), jnp.float32),
            pltpu.VMEM((CHUNK, D), jnp.float32),
            pltpu.SemaphoreType.DMA,
        ],
        compiler_params=pltpu.CompilerParams(use_tc_tiling_on_sc=False),
    )
    return f(value.reshape(shape3), depth.reshape(shape3), p0.reshape(shape3),
             p1.reshape(shape3), p2.reshape(shape3),
             src_table, depth_table, sp0, sp1, sp2)


def kernel(value, depth, position, src_table, depth_table, spatial_tables):
    B, S = value.shape
    out = _run(value, depth,
               position[:, :, 0], position[:, :, 1], position[:, :, 2],
               src_table, depth_table,
               spatial_tables[0], spatial_tables[1], spatial_tables[2])
    return out.reshape(B, S, D)


# speedup vs baseline: 1.5743x; 1.0297x over previous
import jax
import jax.numpy as jnp
from jax import lax
from jax.experimental import pallas as pl
from jax.experimental.pallas import tpu as pltpu
from jax.experimental.pallas import tpu_sc as plsc

NC = 2
NS = 16
NW = NC * NS
D = 64
CHUNK = 128


def _body(value, depth, p0, p1, p2, src_tbl, dep_tbl, sp0, sp1, sp2, out,
          iv, idp, i0, i1, i2, acc, sem):
    wid = lax.axis_index("s") * NC + lax.axis_index("c")
    n_chunks = iv.shape[0]

    pltpu.sync_copy(value.at[wid], iv)
    pltpu.sync_copy(depth.at[wid], idp)
    pltpu.sync_copy(p0.at[wid], i0)
    pltpu.sync_copy(p1.at[wid], i1)
    pltpu.sync_copy(p2.at[wid], i2)

    src_descs = []
    for k in range(n_chunks):
        dst = acc.at[pl.ds(k * CHUNK, CHUNK)]
        src_descs.append(pltpu.async_copy(src_tbl.at[iv.at[k]], dst, sem))
    for d in src_descs:
        d.wait()

    add_descs = []
    for k in range(n_chunks):
        dst = acc.at[pl.ds(k * CHUNK, CHUNK)]
        add_descs.append(
            pltpu.async_copy(dep_tbl.at[idp.at[k]], dst, sem, add=True))
        add_descs.append(
            pltpu.async_copy(sp0.at[i0.at[k]], dst, sem, add=True))
        add_descs.append(
            pltpu.async_copy(sp1.at[i1.at[k]], dst, sem, add=True))
        add_descs.append(
            pltpu.async_copy(sp2.at[i2.at[k]], dst, sem, add=True))
    for d in add_descs:
        d.wait()

    pltpu.sync_copy(acc, out.at[pl.ds(wid * n_chunks * CHUNK,
                                      n_chunks * CHUNK)])


@jax.jit
def _run(value, depth, p0, p1, p2, src_table, depth_table, sp0, sp1, sp2):
    n_tok = value.size
    n_chunks = n_tok // (NW * CHUNK)
    shape3 = (NW, n_chunks, CHUNK)
    mesh = plsc.VectorSubcoreMesh(core_axis_name="c", subcore_axis_name="s")
    f = pl.kernel(
        _body,
        out_type=jax.ShapeDtypeStruct((n_tok, D), jnp.float32),
        mesh=mesh,
        scratch_types=[
            pltpu.VMEM((n_chunks, CHUNK), jnp.int32),
            pltpu.VMEM((n_chunks, CHUNK), jnp.int32),
            pltpu.VMEM((n_chunks, CHUNK), jnp.int32),
            pltpu.VMEM((n_chunks, CHUNK), jnp.int32),
            pltpu.VMEM((n_chunks, CHUNK), jnp.int32),
            pltpu.VMEM((n_chunks * CHUNK, D), jnp.float32),
            pltpu.SemaphoreType.DMA,
        ],
        compiler_params=pltpu.CompilerParams(use_tc_tiling_on_sc=False),
    )
    return f(value.reshape(shape3), depth.reshape(shape3), p0.reshape(shape3),
             p1.reshape(shape3), p2.reshape(shape3),
             src_table, depth_table, sp0, sp1, sp2)


def kernel(value, depth, position, src_table, depth_table, spatial_tables):
    B, S = value.shape
    out = _run(value, depth,
               position[:, :, 0], position[:, :, 1], position[:, :, 2],
               src_table, depth_table,
               spatial_tables[0], spatial_tables[1], spatial_tables[2])
    return out.reshape(B, S, D)


# speedup vs baseline: 3.4989x; 2.2225x over previous
import jax
import jax.numpy as jnp
from jax import lax
from jax.experimental import pallas as pl
from jax.experimental.pallas import tpu as pltpu
from jax.experimental.pallas import tpu_sc as plsc

NC = 2
NS = 16
NW = NC * NS
D = 64
CHUNK = 128
GRP = 16


def _body(value, dp, p1, p2, src_tbl, tdp, sp1, sp2, out,
          iv, idp, i1, i2, tdp_v, sp1_v, sp2_v, acc, sem):
    wid = lax.axis_index("s") * NC + lax.axis_index("c")
    n_chunks = iv.shape[0]

    pltpu.sync_copy(value.at[wid], iv)
    pltpu.sync_copy(dp.at[wid], idp)
    pltpu.sync_copy(p1.at[wid], i1)
    pltpu.sync_copy(p2.at[wid], i2)
    pltpu.sync_copy(tdp, tdp_v)
    pltpu.sync_copy(sp1, sp1_v)
    pltpu.sync_copy(sp2, sp2_v)

    cols = jnp.arange(GRP, dtype=jnp.int32)
    descs = []
    for k in range(n_chunks):
        @pl.loop(0, CHUNK // GRP)
        def _grp(g, k=k):
            dpv = idp[k, pl.ds(g * GRP, GRP)]
            p1v = i1[k, pl.ds(g * GRP, GRP)]
            p2v = i2[k, pl.ds(g * GRP, GRP)]
            base_t = k * CHUNK + g * GRP
            for l in range(GRP):
                sel = jnp.full((GRP,), l, jnp.int32)
                r_dp = dpv.at[sel].get(mode="promise_in_bounds")
                r_p1 = p1v.at[sel].get(mode="promise_in_bounds")
                r_p2 = p2v.at[sel].get(mode="promise_in_bounds")
                for c in range(D // GRP):
                    colv = cols + (GRP * c)
                    x = (plsc.load_gather(tdp_v, [r_dp, colv])
                         + plsc.load_gather(sp1_v, [r_p1, colv])) \
                        + plsc.load_gather(sp2_v, [r_p2, colv])
                    acc[base_t + l, pl.ds(GRP * c, GRP)] = x

        dst = acc.at[pl.ds(k * CHUNK, CHUNK)]
        descs.append(
            pltpu.async_copy(src_tbl.at[iv.at[k]], dst, sem, add=True))

    for d in descs:
        d.wait()

    pltpu.sync_copy(acc, out.at[pl.ds(wid * n_chunks * CHUNK,
                                      n_chunks * CHUNK)])


@jax.jit
def _run(value, dp, p1, p2, src_table, tdp, sp1, sp2):
    n_tok = value.size
    n_chunks = n_tok // (NW * CHUNK)
    shape3 = (NW, n_chunks, CHUNK)
    mesh = plsc.VectorSubcoreMesh(core_axis_name="c", subcore_axis_name="s")
    f = pl.kernel(
        _body,
        out_type=jax.ShapeDtypeStruct((n_tok, D), jnp.float32),
        mesh=mesh,
        scratch_types=[
            pltpu.VMEM((n_chunks, CHUNK), jnp.int32),
            pltpu.VMEM((n_chunks, CHUNK), jnp.int32),
            pltpu.VMEM((n_chunks, CHUNK), jnp.int32),
            pltpu.VMEM((n_chunks, CHUNK), jnp.int32),
            pltpu.VMEM(tdp.shape, jnp.float32),
            pltpu.VMEM(sp1.shape, jnp.float32),
            pltpu.VMEM(sp2.shape, jnp.float32),
            pltpu.VMEM((n_chunks * CHUNK, D), jnp.float32),
            pltpu.SemaphoreType.DMA,
        ],
        compiler_params=pltpu.CompilerParams(use_tc_tiling_on_sc=False, needs_layout_passes=False),
    )
    return f(value.reshape(shape3), dp.reshape(shape3), p1.reshape(shape3),
             p2.reshape(shape3), src_table, tdp, sp1, sp2)


def kernel(value, depth, position, src_table, depth_table, spatial_tables):
    B, S = value.shape
    tdp = (depth_table[:, None, :] + spatial_tables[0][None, :, :]).reshape(
        -1, D)
    dp_idx = depth * 64 + position[:, :, 0]
    out = _run(value, dp_idx, position[:, :, 1], position[:, :, 2],
               src_table, tdp, spatial_tables[1], spatial_tables[2])
    return out.reshape(B, S, D)
